# baseline (speedup 1.0000x reference)
import jax
import jax.numpy as jnp
from jax import lax
from jax.experimental import pallas as pl
from jax.experimental.pallas import tpu as pltpu
from jax.experimental.pallas import tpu_sc as plsc

N_NODES = 10000
N_EDGES = 320000
D = 128
DH = D // 2

NC = 2
NS = 16
CHUNK = 128
STEPS = 160
EDGES_PER_TILE = STEPS * CHUNK
E_PAD = NS * EDGES_PER_TILE
DUMMY_ROW = 10200
CNT_STEPS = STEPS // NC
NPAD = 10240
ROWS_PER_TILE = NPAD // NS
ZROWS = 160


def _sc_accumulate(rows_hbm, cols2_hbm, xs_hbm, sum_hbm, cnt0_hbm, cnt1_hbm,
                   ridx_v, cidx_v, feat0, feat1, ones_v, zrow_v, zcnt_v,
                   ssum, scnt, sem0, sem1):
    c = lax.axis_index("c")
    s = lax.axis_index("s")

    def fill_z(i, carry):
        for j in range(DH // 16):
            zrow_v[i, pl.ds(j * 16, 16)] = jnp.zeros((16,), jnp.float32)
        zcnt_v[i, :] = jnp.zeros((16,), jnp.float32)
        return carry
    lax.fori_loop(0, ZROWS, fill_z, 0)

    def fill_o(i, carry):
        ones_v[i, :] = jnp.full((16,), 1.0, jnp.float32)
        return carry
    lax.fori_loop(0, CHUNK, fill_o, 0)

    pltpu.sync_copy(rows_hbm.at[s], ridx_v)
    pltpu.sync_copy(cols2_hbm.at[c, s], cidx_v)

    rbase = s * ROWS_PER_TILE
    for k in range(ROWS_PER_TILE // ZROWS):
        pltpu.sync_copy(zrow_v, ssum.at[pl.ds(rbase + k * ZROWS, ZROWS)])
        pltpu.sync_copy(zcnt_v, scnt.at[pl.ds(rbase + k * ZROWS, ZROWS)])
    plsc.subcore_barrier()

    xtab = xs_hbm
    bufs = (feat0, feat1)
    sems = (sem0, sem1)

    pltpu.async_copy(xtab.at[cidx_v.at[0]], feat0, sem0)
    pltpu.async_copy(xtab.at[cidx_v.at[1]], feat1, sem1)

    def step(i, carry):
        for k in range(2):
            t = 2 * i + k
            buf, sem = bufs[k], sems[k]
            pltpu.make_async_copy(xtab.at[pl.ds(0, CHUNK)], buf, sem).wait()
            pltpu.sync_copy(buf, ssum.at[ridx_v.at[t]], add=True)

            @pl.when(t // CNT_STEPS == c)
            def _count():
                pltpu.sync_copy(ones_v, scnt.at[ridx_v.at[t]], add=True)

            @pl.when(t + 2 < STEPS)
            def _prefetch():
                pltpu.async_copy(xtab.at[cidx_v.at[t + 2]], buf, sem)
        return carry
    lax.fori_loop(0, STEPS // 2, step, 0)

    plsc.subcore_barrier()

    @pl.when(c == 0)
    def _out0():
        pltpu.sync_copy(ssum.at[pl.ds(rbase, ROWS_PER_TILE)],
                        sum_hbm.at[pl.ds(rbase, ROWS_PER_TILE),
                                   pl.ds(0, DH)])
        pltpu.sync_copy(scnt.at[pl.ds(rbase, ROWS_PER_TILE)],
                        cnt0_hbm.at[pl.ds(rbase, ROWS_PER_TILE)])

    @pl.when(c == 1)
    def _out1():
        pltpu.sync_copy(ssum.at[pl.ds(rbase, ROWS_PER_TILE)],
                        sum_hbm.at[pl.ds(rbase, ROWS_PER_TILE),
                                   pl.ds(DH, DH)])
        pltpu.sync_copy(scnt.at[pl.ds(rbase, ROWS_PER_TILE)],
                        cnt1_hbm.at[pl.ds(rbase, ROWS_PER_TILE)])


def _tc_dense(x_ref, s_ref, c0_ref, c1_ref, w1_ref, w2_ref, b_ref, o_ref):
    cnt = c0_ref[...][:, 0:1] + c1_ref[...][:, 0:1]
    inv = 1.0 / (cnt + 1e-8)
    acc = jnp.dot(x_ref[...], w1_ref[...], preferred_element_type=jnp.float32)
    acc = acc + jnp.dot(s_ref[...] * inv, w2_ref[...],
                        preferred_element_type=jnp.float32)
    o_ref[...] = jnp.maximum(acc + b_ref[...], 0.0)


@jax.jit
def kernel(x, edge_index, W, b):
    ei = edge_index.astype(jnp.int32)
    pad = E_PAD - N_EDGES
    rows = jnp.pad(ei[0], (0, pad), constant_values=DUMMY_ROW)
    rows = rows.reshape(NS, STEPS, CHUNK)
    c2 = jnp.pad(ei[1] * 2, (0, pad), constant_values=0)
    cols2 = jnp.stack([c2, c2 + 1]).reshape(NC, NS, STEPS, CHUNK)
    xs = x.reshape(2 * N_NODES, DH)

    mesh = plsc.VectorSubcoreMesh(core_axis_name="c", subcore_axis_name="s")
    sc = pl.kernel(
        _sc_accumulate,
        out_type=(
            jax.ShapeDtypeStruct((NPAD, D), jnp.float32),
            jax.ShapeDtypeStruct((NPAD, 16), jnp.float32),
            jax.ShapeDtypeStruct((NPAD, 16), jnp.float32),
        ),
        mesh=mesh,
        scratch_types=[
            pltpu.VMEM((STEPS, CHUNK), jnp.int32),
            pltpu.VMEM((STEPS, CHUNK), jnp.int32),
            pltpu.VMEM((CHUNK, DH), jnp.float32),
            pltpu.VMEM((CHUNK, DH), jnp.float32),
            pltpu.VMEM((CHUNK, 16), jnp.float32),
            pltpu.VMEM((ZROWS, DH), jnp.float32),
            pltpu.VMEM((ZROWS, 16), jnp.float32),
            pltpu.VMEM_SHARED((NPAD, DH), jnp.float32),
            pltpu.VMEM_SHARED((NPAD, 16), jnp.float32),
            pltpu.SemaphoreType.DMA,
            pltpu.SemaphoreType.DMA,
        ],
        compiler_params=pltpu.CompilerParams(use_tc_tiling_on_sc=False),
    )
    sum_p, cnt0, cnt1 = sc(rows, cols2, xs)

    wt = W.T
    w1 = wt[:D]
    w2 = wt[D:]
    b2 = b.reshape(1, -1)

    blk = 1000
    out = pl.pallas_call(
        _tc_dense,
        grid=(N_NODES // blk,),
        in_specs=[
            pl.BlockSpec((blk, D), lambda i: (i, 0)),
            pl.BlockSpec((blk, D), lambda i: (i, 0)),
            pl.BlockSpec((blk, 16), lambda i: (i, 0)),
            pl.BlockSpec((blk, 16), lambda i: (i, 0)),
            pl.BlockSpec((D, D), lambda i: (0, 0)),
            pl.BlockSpec((D, D), lambda i: (0, 0)),
            pl.BlockSpec((1, D), lambda i: (0, 0)),
        ],
        out_specs=pl.BlockSpec((blk, D), lambda i: (i, 0)),
        out_shape=jax.ShapeDtypeStruct((N_NODES, D), jnp.float32),
    )(x, sum_p, cnt0, cnt1, w1, w2, b2)
    return out


# speedup vs baseline: 1.0008x; 1.0008x over previous
import jax
import jax.numpy as jnp
from jax import lax
from jax.experimental import pallas as pl
from jax.experimental.pallas import tpu as pltpu
from jax.experimental.pallas import tpu_sc as plsc

N_NODES = 10000
N_EDGES = 320000
D = 128
DH = D // 2

NC = 2
NS = 16
CHUNK = 128
STEPS = 160
EDGES_PER_TILE = STEPS * CHUNK
E_PAD = NS * EDGES_PER_TILE
DUMMY_ROW = 10000
CNT_STEPS = STEPS // NC
NPAD = 10240
ROWS_PER_TILE = NPAD // NS
ZROWS = 160


def _sc_accumulate(rows_hbm, cols2_hbm, xs_hbm, sum_hbm, cnt0_hbm, cnt1_hbm,
                   ridx_v, cidx_v, feat0, feat1, ones_v, zrow_v, zcnt_v,
                   ssum, scnt, sem0, sem1):
    c = lax.axis_index("c")
    s = lax.axis_index("s")

    def fill_z(i, carry):
        for j in range(DH // 16):
            zrow_v[i, pl.ds(j * 16, 16)] = jnp.zeros((16,), jnp.float32)
        zcnt_v[i, :] = jnp.zeros((16,), jnp.float32)
        return carry
    lax.fori_loop(0, ZROWS, fill_z, 0)

    def fill_o(i, carry):
        ones_v[i, :] = jnp.full((16,), 1.0, jnp.float32)
        return carry
    lax.fori_loop(0, CHUNK, fill_o, 0)

    pltpu.sync_copy(rows_hbm.at[s], ridx_v)
    pltpu.sync_copy(cols2_hbm.at[c, s], cidx_v)

    rbase = s * ROWS_PER_TILE
    for k in range(ROWS_PER_TILE // ZROWS):
        pltpu.sync_copy(zrow_v, ssum.at[pl.ds(rbase + k * ZROWS, ZROWS)])
        pltpu.sync_copy(zcnt_v, scnt.at[pl.ds(rbase + k * ZROWS, ZROWS)])
    plsc.subcore_barrier()

    xtab = xs_hbm
    bufs = (feat0, feat1)
    sems = (sem0, sem1)

    pltpu.async_copy(xtab.at[cidx_v.at[0]], feat0, sem0)
    pltpu.async_copy(xtab.at[cidx_v.at[1]], feat1, sem1)

    def step(i, carry):
        for k in range(2):
            t = 2 * i + k
            buf, sem = bufs[k], sems[k]
            pltpu.make_async_copy(xtab.at[pl.ds(0, CHUNK)], buf, sem).wait()
            pltpu.sync_copy(buf, ssum.at[ridx_v.at[t]], add=True)

            @pl.when(t // CNT_STEPS == c)
            def _count():
                pltpu.sync_copy(ones_v, scnt.at[ridx_v.at[t]], add=True)

            @pl.when(t + 2 < STEPS)
            def _prefetch():
                pltpu.async_copy(xtab.at[cidx_v.at[t + 2]], buf, sem)
        return carry
    lax.fori_loop(0, STEPS // 2, step, 0)

    plsc.subcore_barrier()

    @pl.when(c == 0)
    def _out0():
        pltpu.sync_copy(ssum.at[pl.ds(rbase, ROWS_PER_TILE)],
                        sum_hbm.at[pl.ds(rbase, ROWS_PER_TILE),
                                   pl.ds(0, DH)])
        pltpu.sync_copy(scnt.at[pl.ds(rbase, ROWS_PER_TILE)],
                        cnt0_hbm.at[pl.ds(rbase, ROWS_PER_TILE)])

    @pl.when(c == 1)
    def _out1():
        pltpu.sync_copy(ssum.at[pl.ds(rbase, ROWS_PER_TILE)],
                        sum_hbm.at[pl.ds(rbase, ROWS_PER_TILE),
                                   pl.ds(DH, DH)])
        pltpu.sync_copy(scnt.at[pl.ds(rbase, ROWS_PER_TILE)],
                        cnt1_hbm.at[pl.ds(rbase, ROWS_PER_TILE)])


def _tc_dense(x_ref, s_ref, c0_ref, c1_ref, w1_ref, w2_ref, b_ref, o_ref):
    cnt = c0_ref[...][:, 0:1] + c1_ref[...][:, 0:1]
    inv = 1.0 / (cnt + 1e-8)
    acc = jnp.dot(x_ref[...], w1_ref[...], preferred_element_type=jnp.float32)
    acc = acc + jnp.dot(s_ref[...] * inv, w2_ref[...],
                        preferred_element_type=jnp.float32)
    o_ref[...] = jnp.maximum(acc + b_ref[...], 0.0)


@jax.jit
def kernel(x, edge_index, W, b):
    ei = edge_index.astype(jnp.int32)
    pad = E_PAD - N_EDGES
    dummy = DUMMY_ROW + jnp.arange(pad, dtype=jnp.int32) % (NPAD - DUMMY_ROW)
    rows = jnp.concatenate([ei[0], dummy]).reshape(NS, STEPS, CHUNK)
    c2 = jnp.pad(ei[1] * 2, (0, pad), constant_values=0)
    cols2 = jnp.stack([c2, c2 + 1]).reshape(NC, NS, STEPS, CHUNK)
    xs = x.reshape(2 * N_NODES, DH)

    mesh = plsc.VectorSubcoreMesh(core_axis_name="c", subcore_axis_name="s")
    sc = pl.kernel(
        _sc_accumulate,
        out_type=(
            jax.ShapeDtypeStruct((NPAD, D), jnp.float32),
            jax.ShapeDtypeStruct((NPAD, 16), jnp.float32),
            jax.ShapeDtypeStruct((NPAD, 16), jnp.float32),
        ),
        mesh=mesh,
        scratch_types=[
            pltpu.VMEM((STEPS, CHUNK), jnp.int32),
            pltpu.VMEM((STEPS, CHUNK), jnp.int32),
            pltpu.VMEM((CHUNK, DH), jnp.float32),
            pltpu.VMEM((CHUNK, DH), jnp.float32),
            pltpu.VMEM((CHUNK, 16), jnp.float32),
            pltpu.VMEM((ZROWS, DH), jnp.float32),
            pltpu.VMEM((ZROWS, 16), jnp.float32),
            pltpu.VMEM_SHARED((NPAD, DH), jnp.float32),
            pltpu.VMEM_SHARED((NPAD, 16), jnp.float32),
            pltpu.SemaphoreType.DMA,
            pltpu.SemaphoreType.DMA,
        ],
        compiler_params=pltpu.CompilerParams(use_tc_tiling_on_sc=False),
    )
    sum_p, cnt0, cnt1 = sc(rows, cols2, xs)

    wt = W.T
    w1 = wt[:D]
    w2 = wt[D:]
    b2 = b.reshape(1, -1)

    blk = 1000
    out = pl.pallas_call(
        _tc_dense,
        grid=(N_NODES // blk,),
        in_specs=[
            pl.BlockSpec((blk, D), lambda i: (i, 0)),
            pl.BlockSpec((blk, D), lambda i: (i, 0)),
            pl.BlockSpec((blk, 16), lambda i: (i, 0)),
            pl.BlockSpec((blk, 16), lambda i: (i, 0)),
            pl.BlockSpec((D, D), lambda i: (0, 0)),
            pl.BlockSpec((D, D), lambda i: (0, 0)),
            pl.BlockSpec((1, D), lambda i: (0, 0)),
        ],
        out_specs=pl.BlockSpec((blk, D), lambda i: (i, 0)),
        out_shape=jax.ShapeDtypeStruct((N_NODES, D), jnp.float32),
    )(x, sum_p, cnt0, cnt1, w1, w2, b2)
    return out


# speedup vs baseline: 2.3733x; 2.3715x over previous
import jax
import jax.numpy as jnp
from jax import lax
from jax.experimental import pallas as pl
from jax.experimental.pallas import tpu as pltpu
from jax.experimental.pallas import tpu_sc as plsc

N_NODES = 10000
N_EDGES = 320000
D = 128
DH = D // 2

NC = 2
NS = 16
CHUNK = 125
STEPS = 160
EDGES_PER_TILE = STEPS * CHUNK
E_PAD = NS * EDGES_PER_TILE
DUMMY_ROW = 10000
CNT_STEPS = STEPS // NC
NPAD = 10240
ROWS_PER_TILE = NPAD // NS
ZROWS = 160


def _sc_accumulate(rows_hbm, cols2_hbm, xs_hbm, sum_hbm, cnt0_hbm, cnt1_hbm,
                   ridx_v, cidx_v, feat0, feat1, ones_v, zrow_v, zcnt_v,
                   ssum, scnt, sem0, sem1):
    c = lax.axis_index("c")
    s = lax.axis_index("s")

    def fill_z(i, carry):
        for j in range(DH // 16):
            zrow_v[i, pl.ds(j * 16, 16)] = jnp.zeros((16,), jnp.float32)
        zcnt_v[i, :] = jnp.zeros((16,), jnp.float32)
        return carry
    lax.fori_loop(0, ZROWS, fill_z, 0)

    def fill_o(i, carry):
        ones_v[i, :] = jnp.full((16,), 1.0, jnp.float32)
        return carry
    lax.fori_loop(0, CHUNK, fill_o, 0)

    pltpu.sync_copy(rows_hbm.at[s], ridx_v)
    pltpu.sync_copy(cols2_hbm.at[c, s], cidx_v)

    rbase = s * ROWS_PER_TILE
    for k in range(ROWS_PER_TILE // ZROWS):
        pltpu.sync_copy(zrow_v, ssum.at[pl.ds(rbase + k * ZROWS, ZROWS)])
        pltpu.sync_copy(zcnt_v, scnt.at[pl.ds(rbase + k * ZROWS, ZROWS)])
    plsc.subcore_barrier()

    xtab = xs_hbm
    bufs = (feat0, feat1)
    sems = (sem0, sem1)

    pltpu.async_copy(xtab.at[cidx_v.at[0]], feat0, sem0)
    pltpu.async_copy(xtab.at[cidx_v.at[1]], feat1, sem1)

    def step(i, carry):
        for k in range(2):
            t = 2 * i + k
            buf, sem = bufs[k], sems[k]
            pltpu.make_async_copy(xtab.at[pl.ds(0, CHUNK)], buf, sem).wait()
            pltpu.sync_copy(buf, ssum.at[ridx_v.at[t]], add=True)

            @pl.when(t // CNT_STEPS == c)
            def _count():
                pltpu.sync_copy(ones_v, scnt.at[ridx_v.at[t]], add=True)

            @pl.when(t + 2 < STEPS)
            def _prefetch():
                pltpu.async_copy(xtab.at[cidx_v.at[t + 2]], buf, sem)
        return carry
    lax.fori_loop(0, STEPS // 2, step, 0)

    plsc.subcore_barrier()

    @pl.when(c == 0)
    def _out0():
        pltpu.sync_copy(ssum.at[pl.ds(rbase, ROWS_PER_TILE)],
                        sum_hbm.at[pl.ds(rbase, ROWS_PER_TILE),
                                   pl.ds(0, DH)])
        pltpu.sync_copy(scnt.at[pl.ds(rbase, ROWS_PER_TILE)],
                        cnt0_hbm.at[pl.ds(rbase, ROWS_PER_TILE)])

    @pl.when(c == 1)
    def _out1():
        pltpu.sync_copy(ssum.at[pl.ds(rbase, ROWS_PER_TILE)],
                        sum_hbm.at[pl.ds(rbase, ROWS_PER_TILE),
                                   pl.ds(DH, DH)])
        pltpu.sync_copy(scnt.at[pl.ds(rbase, ROWS_PER_TILE)],
                        cnt1_hbm.at[pl.ds(rbase, ROWS_PER_TILE)])


def _tc_dense(x_ref, s_ref, c0_ref, c1_ref, w1_ref, w2_ref, b_ref, o_ref):
    cnt = c0_ref[...][:, 0:1] + c1_ref[...][:, 0:1]
    inv = 1.0 / (cnt + 1e-8)
    acc = jnp.dot(x_ref[...], w1_ref[...], preferred_element_type=jnp.float32)
    acc = acc + jnp.dot(s_ref[...] * inv, w2_ref[...],
                        preferred_element_type=jnp.float32)
    o_ref[...] = jnp.maximum(acc + b_ref[...], 0.0)


@jax.jit
def kernel(x, edge_index, W, b):
    ei = edge_index.astype(jnp.int32)
    rows = ei[0].reshape(NS, STEPS, CHUNK)
    c2 = ei[1] * 2
    cols2 = jnp.stack([c2, c2 + 1]).reshape(NC, NS, STEPS, CHUNK)
    xs = x.reshape(2 * N_NODES, DH)

    mesh = plsc.VectorSubcoreMesh(core_axis_name="c", subcore_axis_name="s")
    sc = pl.kernel(
        _sc_accumulate,
        out_type=(
            jax.ShapeDtypeStruct((NPAD, D), jnp.float32),
            jax.ShapeDtypeStruct((NPAD, 16), jnp.float32),
            jax.ShapeDtypeStruct((NPAD, 16), jnp.float32),
        ),
        mesh=mesh,
        scratch_types=[
            pltpu.VMEM((STEPS, CHUNK), jnp.int32),
            pltpu.VMEM((STEPS, CHUNK), jnp.int32),
            pltpu.VMEM((CHUNK, DH), jnp.float32),
            pltpu.VMEM((CHUNK, DH), jnp.float32),
            pltpu.VMEM((CHUNK, 16), jnp.float32),
            pltpu.VMEM((ZROWS, DH), jnp.float32),
            pltpu.VMEM((ZROWS, 16), jnp.float32),
            pltpu.VMEM_SHARED((NPAD, DH), jnp.float32),
            pltpu.VMEM_SHARED((NPAD, 16), jnp.float32),
            pltpu.SemaphoreType.DMA,
            pltpu.SemaphoreType.DMA,
        ],
        compiler_params=pltpu.CompilerParams(use_tc_tiling_on_sc=False),
    )
    sum_p, cnt0, cnt1 = sc(rows, cols2, xs)

    wt = W.T
    w1 = wt[:D]
    w2 = wt[D:]
    b2 = b.reshape(1, -1)

    blk = 1000
    out = pl.pallas_call(
        _tc_dense,
        grid=(N_NODES // blk,),
        in_specs=[
            pl.BlockSpec((blk, D), lambda i: (i, 0)),
            pl.BlockSpec((blk, D), lambda i: (i, 0)),
            pl.BlockSpec((blk, 16), lambda i: (i, 0)),
            pl.BlockSpec((blk, 16), lambda i: (i, 0)),
            pl.BlockSpec((D, D), lambda i: (0, 0)),
            pl.BlockSpec((D, D), lambda i: (0, 0)),
            pl.BlockSpec((1, D), lambda i: (0, 0)),
        ],
        out_specs=pl.BlockSpec((blk, D), lambda i: (i, 0)),
        out_shape=jax.ShapeDtypeStruct((N_NODES, D), jnp.float32),
    )(x, sum_p, cnt0, cnt1, w1, w2, b2)
    return out


# speedup vs baseline: 2.4279x; 1.0230x over previous
import jax
import jax.numpy as jnp
from jax import lax
from jax.experimental import pallas as pl
from jax.experimental.pallas import tpu as pltpu
from jax.experimental.pallas import tpu_sc as plsc

N_NODES = 10000
N_EDGES = 320000
D = 128
DH = D // 2

NC = 2
NS = 16
CHUNK = 125
STEPS = 160
CNT_STEPS = STEPS // NC
NPAD = 10240
ROWS_PER_TILE = NPAD // NS
ZROWS = 160


def _sc_accumulate(rows_hbm, cols2_hbm, xs_hbm, sum_hbm, cnt0_hbm, cnt1_hbm,
                   ridx_v, cidx_v, feat0, feat1, ones_v, zrow_v, zcnt_v,
                   ssum, scnt, sem0, sem1, csem):
    c = lax.axis_index("c")
    s = lax.axis_index("s")

    def fill_z(i, carry):
        for j in range(DH // 16):
            zrow_v[i, pl.ds(j * 16, 16)] = jnp.zeros((16,), jnp.float32)
        zcnt_v[i, :] = jnp.zeros((16,), jnp.float32)
        return carry
    lax.fori_loop(0, ZROWS, fill_z, 0)

    def fill_o(i, carry):
        ones_v[i, :] = jnp.full((16,), 1.0, jnp.float32)
        return carry
    lax.fori_loop(0, CHUNK, fill_o, 0)

    pltpu.sync_copy(rows_hbm.at[s], ridx_v)
    pltpu.sync_copy(cols2_hbm.at[c, s], cidx_v)

    rbase = s * ROWS_PER_TILE
    for k in range(ROWS_PER_TILE // ZROWS):
        pltpu.sync_copy(zrow_v, ssum.at[pl.ds(rbase + k * ZROWS, ZROWS)])
        pltpu.sync_copy(zcnt_v, scnt.at[pl.ds(rbase + k * ZROWS, ZROWS)])
    plsc.subcore_barrier()

    xtab = xs_hbm
    bufs = (feat0, feat1)
    sems = (sem0, sem1)

    pltpu.async_copy(xtab.at[cidx_v.at[0]], feat0, sem0)
    pltpu.async_copy(xtab.at[cidx_v.at[1]], feat1, sem1)

    def step(i, carry):
        for k in range(2):
            t = 2 * i + k
            buf, sem = bufs[k], sems[k]
            pltpu.make_async_copy(xtab.at[pl.ds(0, CHUNK)], buf, sem).wait()
            pltpu.sync_copy(buf, ssum.at[ridx_v.at[t]], add=True)

            @pl.when(t // CNT_STEPS == c)
            def _count():
                pltpu.async_copy(ones_v, scnt.at[ridx_v.at[t]], csem,
                                 add=True)

            @pl.when(t + 2 < STEPS)
            def _prefetch():
                pltpu.async_copy(xtab.at[cidx_v.at[t + 2]], buf, sem)
        return carry
    lax.fori_loop(0, STEPS // 2, step, 0)

    def drain_cnt(i, carry):
        pltpu.make_async_copy(ones_v, scnt.at[ridx_v.at[0]], csem).wait()
        return carry
    lax.fori_loop(0, CNT_STEPS, drain_cnt, 0)

    plsc.subcore_barrier()

    @pl.when(c == 0)
    def _out0():
        pltpu.sync_copy(ssum.at[pl.ds(rbase, ROWS_PER_TILE)],
                        sum_hbm.at[pl.ds(rbase, ROWS_PER_TILE),
                                   pl.ds(0, DH)])
        pltpu.sync_copy(scnt.at[pl.ds(rbase, ROWS_PER_TILE)],
                        cnt0_hbm.at[pl.ds(rbase, ROWS_PER_TILE)])

    @pl.when(c == 1)
    def _out1():
        pltpu.sync_copy(ssum.at[pl.ds(rbase, ROWS_PER_TILE)],
                        sum_hbm.at[pl.ds(rbase, ROWS_PER_TILE),
                                   pl.ds(DH, DH)])
        pltpu.sync_copy(scnt.at[pl.ds(rbase, ROWS_PER_TILE)],
                        cnt1_hbm.at[pl.ds(rbase, ROWS_PER_TILE)])


def _tc_xw1(x_ref, w1_ref, b_ref, y_ref):
    y_ref[...] = jnp.dot(x_ref[...], w1_ref[...],
                         preferred_element_type=jnp.float32) + b_ref[...]


def _tc_tail(y_ref, s_ref, c0_ref, c1_ref, w2_ref, o_ref):
    cnt = c0_ref[...][:, 0:1] + c1_ref[...][:, 0:1]
    inv = 1.0 / (cnt + 1e-8)
    acc = y_ref[...] + jnp.dot(s_ref[...] * inv, w2_ref[...],
                               preferred_element_type=jnp.float32)
    o_ref[...] = jnp.maximum(acc, 0.0)


@jax.jit
def kernel(x, edge_index, W, b):
    ei = edge_index.astype(jnp.int32)
    rows = ei[0].reshape(NS, STEPS, CHUNK)
    c2 = ei[1] * 2
    cols2 = jnp.stack([c2, c2 + 1]).reshape(NC, NS, STEPS, CHUNK)
    xs = x.reshape(2 * N_NODES, DH)

    wt = W.T
    w1 = wt[:D]
    w2 = wt[D:]
    b2 = b.reshape(1, -1)

    blk = 1000
    y1 = pl.pallas_call(
        _tc_xw1,
        grid=(N_NODES // blk,),
        in_specs=[
            pl.BlockSpec((blk, D), lambda i: (i, 0)),
            pl.BlockSpec((D, D), lambda i: (0, 0)),
            pl.BlockSpec((1, D), lambda i: (0, 0)),
        ],
        out_specs=pl.BlockSpec((blk, D), lambda i: (i, 0)),
        out_shape=jax.ShapeDtypeStruct((N_NODES, D), jnp.float32),
    )(x, w1, b2)

    mesh = plsc.VectorSubcoreMesh(core_axis_name="c", subcore_axis_name="s")
    sc = pl.kernel(
        _sc_accumulate,
        out_type=(
            jax.ShapeDtypeStruct((NPAD, D), jnp.float32),
            jax.ShapeDtypeStruct((NPAD, 16), jnp.float32),
            jax.ShapeDtypeStruct((NPAD, 16), jnp.float32),
        ),
        mesh=mesh,
        scratch_types=[
            pltpu.VMEM((STEPS, CHUNK), jnp.int32),
            pltpu.VMEM((STEPS, CHUNK), jnp.int32),
            pltpu.VMEM((CHUNK, DH), jnp.float32),
            pltpu.VMEM((CHUNK, DH), jnp.float32),
            pltpu.VMEM((CHUNK, 16), jnp.float32),
            pltpu.VMEM((ZROWS, DH), jnp.float32),
            pltpu.VMEM((ZROWS, 16), jnp.float32),
            pltpu.VMEM_SHARED((NPAD, DH), jnp.float32),
            pltpu.VMEM_SHARED((NPAD, 16), jnp.float32),
            pltpu.SemaphoreType.DMA,
            pltpu.SemaphoreType.DMA,
            pltpu.SemaphoreType.DMA,
        ],
        compiler_params=pltpu.CompilerParams(use_tc_tiling_on_sc=False),
    )
    sum_p, cnt0, cnt1 = sc(rows, cols2, xs)

    out = pl.pallas_call(
        _tc_tail,
        grid=(N_NODES // blk,),
        in_specs=[
            pl.BlockSpec((blk, D), lambda i: (i, 0)),
            pl.BlockSpec((blk, D), lambda i: (i, 0)),
            pl.BlockSpec((blk, 16), lambda i: (i, 0)),
            pl.BlockSpec((blk, 16), lambda i: (i, 0)),
            pl.BlockSpec((D, D), lambda i: (0, 0)),
        ],
        out_specs=pl.BlockSpec((blk, D), lambda i: (i, 0)),
        out_shape=jax.ShapeDtypeStruct((N_NODES, D), jnp.float32),
    )(y1, sum_p, cnt0, cnt1, w2)
    return out


# speedup vs baseline: 2.6012x; 1.0714x over previous
import jax
import jax.numpy as jnp
from jax import lax
from jax.experimental import pallas as pl
from jax.experimental.pallas import tpu as pltpu
from jax.experimental.pallas import tpu_sc as plsc

N_NODES = 10000
N_EDGES = 320000
D = 128
DH = D // 2

NC = 2
NS = 16
CHUNK = 125
STEPS = 160
CNT_STEPS = STEPS // NC
NPAD = 10240
ROWS_PER_TILE = NPAD // NS
ZROWS = 160


def _sc_accumulate(rows_hbm, cols2_hbm, xs_hbm, sum_hbm, cnt0_hbm, cnt1_hbm,
                   ridx_v, cidx_v, feat0, feat1, ones_v, zrow_v, zcnt_v,
                   ssum, scnt, sem0, sem1, csem):
    c = lax.axis_index("c")
    s = lax.axis_index("s")

    def fill_z(i, carry):
        for j in range(DH // 32):
            zrow_v[i, pl.ds(j * 32, 32)] = jnp.zeros((32,), jnp.bfloat16)
        zcnt_v[i, :] = jnp.zeros((16,), jnp.float32)
        return carry
    lax.fori_loop(0, ZROWS, fill_z, 0)

    def fill_o(i, carry):
        ones_v[i, :] = jnp.full((16,), 1.0, jnp.float32)
        return carry
    lax.fori_loop(0, CHUNK, fill_o, 0)

    pltpu.sync_copy(rows_hbm.at[s], ridx_v)
    pltpu.sync_copy(cols2_hbm.at[c, s], cidx_v)

    rbase = s * ROWS_PER_TILE
    for k in range(ROWS_PER_TILE // ZROWS):
        pltpu.sync_copy(zrow_v, ssum.at[pl.ds(rbase + k * ZROWS, ZROWS)])
        pltpu.sync_copy(zcnt_v, scnt.at[pl.ds(rbase + k * ZROWS, ZROWS)])
    plsc.subcore_barrier()

    xtab = xs_hbm
    bufs = (feat0, feat1)
    sems = (sem0, sem1)

    pltpu.async_copy(xtab.at[cidx_v.at[0]], feat0, sem0)
    pltpu.async_copy(xtab.at[cidx_v.at[1]], feat1, sem1)

    def step(i, carry):
        for k in range(2):
            t = 2 * i + k
            buf, sem = bufs[k], sems[k]
            pltpu.make_async_copy(xtab.at[pl.ds(0, CHUNK)], buf, sem).wait()
            pltpu.sync_copy(buf, ssum.at[ridx_v.at[t]], add=True)

            @pl.when(t // CNT_STEPS == c)
            def _count():
                pltpu.async_copy(ones_v, scnt.at[ridx_v.at[t]], csem,
                                 add=True)

            @pl.when(t + 2 < STEPS)
            def _prefetch():
                pltpu.async_copy(xtab.at[cidx_v.at[t + 2]], buf, sem)
        return carry
    lax.fori_loop(0, STEPS // 2, step, 0)

    def drain_cnt(i, carry):
        pltpu.make_async_copy(ones_v, scnt.at[ridx_v.at[0]], csem).wait()
        return carry
    lax.fori_loop(0, CNT_STEPS, drain_cnt, 0)

    plsc.subcore_barrier()

    @pl.when(c == 0)
    def _out0():
        pltpu.sync_copy(ssum.at[pl.ds(rbase, ROWS_PER_TILE)],
                        sum_hbm.at[pl.ds(rbase, ROWS_PER_TILE),
                                   pl.ds(0, DH)])
        pltpu.sync_copy(scnt.at[pl.ds(rbase, ROWS_PER_TILE)],
                        cnt0_hbm.at[pl.ds(rbase, ROWS_PER_TILE)])

    @pl.when(c == 1)
    def _out1():
        pltpu.sync_copy(ssum.at[pl.ds(rbase, ROWS_PER_TILE)],
                        sum_hbm.at[pl.ds(rbase, ROWS_PER_TILE),
                                   pl.ds(DH, DH)])
        pltpu.sync_copy(scnt.at[pl.ds(rbase, ROWS_PER_TILE)],
                        cnt1_hbm.at[pl.ds(rbase, ROWS_PER_TILE)])


def _tc_xw1(x_ref, w1_ref, b_ref, y_ref):
    y_ref[...] = jnp.dot(x_ref[...], w1_ref[...],
                         preferred_element_type=jnp.float32) + b_ref[...]


def _tc_tail(y_ref, s_ref, c0_ref, c1_ref, w2_ref, o_ref):
    cnt = c0_ref[...][:, 0:1] + c1_ref[...][:, 0:1]
    inv = 1.0 / (cnt + 1e-8)
    mean = s_ref[...].astype(jnp.float32) * inv
    acc = y_ref[...] + jnp.dot(mean, w2_ref[...],
                               preferred_element_type=jnp.float32)
    o_ref[...] = jnp.maximum(acc, 0.0)


@jax.jit
def kernel(x, edge_index, W, b):
    ei = edge_index.astype(jnp.int32)
    rows = ei[0].reshape(NS, STEPS, CHUNK)
    c2 = ei[1] * 2
    cols2 = jnp.stack([c2, c2 + 1]).reshape(NC, NS, STEPS, CHUNK)
    xs = x.astype(jnp.bfloat16).reshape(2 * N_NODES, DH)

    wt = W.T
    w1 = wt[:D]
    w2 = wt[D:]
    b2 = b.reshape(1, -1)

    blk = 1000
    y1 = pl.pallas_call(
        _tc_xw1,
        grid=(N_NODES // blk,),
        in_specs=[
            pl.BlockSpec((blk, D), lambda i: (i, 0)),
            pl.BlockSpec((D, D), lambda i: (0, 0)),
            pl.BlockSpec((1, D), lambda i: (0, 0)),
        ],
        out_specs=pl.BlockSpec((blk, D), lambda i: (i, 0)),
        out_shape=jax.ShapeDtypeStruct((N_NODES, D), jnp.float32),
    )(x, w1, b2)

    mesh = plsc.VectorSubcoreMesh(core_axis_name="c", subcore_axis_name="s")
    sc = pl.kernel(
        _sc_accumulate,
        out_type=(
            jax.ShapeDtypeStruct((NPAD, D), jnp.bfloat16),
            jax.ShapeDtypeStruct((NPAD, 16), jnp.float32),
            jax.ShapeDtypeStruct((NPAD, 16), jnp.float32),
        ),
        mesh=mesh,
        scratch_types=[
            pltpu.VMEM((STEPS, CHUNK), jnp.int32),
            pltpu.VMEM((STEPS, CHUNK), jnp.int32),
            pltpu.VMEM((CHUNK, DH), jnp.bfloat16),
            pltpu.VMEM((CHUNK, DH), jnp.bfloat16),
            pltpu.VMEM((CHUNK, 16), jnp.float32),
            pltpu.VMEM((ZROWS, DH), jnp.bfloat16),
            pltpu.VMEM((ZROWS, 16), jnp.float32),
            pltpu.VMEM_SHARED((NPAD, DH), jnp.bfloat16),
            pltpu.VMEM_SHARED((NPAD, 16), jnp.float32),
            pltpu.SemaphoreType.DMA,
            pltpu.SemaphoreType.DMA,
            pltpu.SemaphoreType.DMA,
        ],
        compiler_params=pltpu.CompilerParams(use_tc_tiling_on_sc=False),
    )
    sum_p, cnt0, cnt1 = sc(rows, cols2, xs)

    out = pl.pallas_call(
        _tc_tail,
        grid=(N_NODES // blk,),
        in_specs=[
            pl.BlockSpec((blk, D), lambda i: (i, 0)),
            pl.BlockSpec((blk, D), lambda i: (i, 0)),
            pl.BlockSpec((blk, 16), lambda i: (i, 0)),
            pl.BlockSpec((blk, 16), lambda i: (i, 0)),
            pl.BlockSpec((D, D), lambda i: (0, 0)),
        ],
        out_specs=pl.BlockSpec((blk, D), lambda i: (i, 0)),
        out_shape=jax.ShapeDtypeStruct((N_NODES, D), jnp.float32),
    )(y1, sum_p, cnt0, cnt1, w2)
    return out


# speedup vs baseline: 2.7851x; 1.0707x over previous
import jax
import jax.numpy as jnp
from jax import lax
from jax.experimental import pallas as pl
from jax.experimental.pallas import tpu as pltpu
from jax.experimental.pallas import tpu_sc as plsc

N_NODES = 10000
N_EDGES = 320000
D = 128
DH = D // 2

NC = 2
NS = 16
CHUNK = 125
STEPS = 160
NBUF = 5
CNT_STEPS = STEPS // NC
NPAD = 10240
ROWS_PER_TILE = NPAD // NS
ZROWS = 160


def _sc_accumulate(rows_hbm, cols2_hbm, xs_hbm, sum_hbm, cnt0_hbm, cnt1_hbm,
                   ridx_v, cidx_v, feats, ones_v, zrow_v, zcnt_v,
                   ssum, scnt, gsems, ssems, csem):
    c = lax.axis_index("c")
    s = lax.axis_index("s")

    def fill_z(i, carry):
        for j in range(DH // 32):
            zrow_v[i, pl.ds(j * 32, 32)] = jnp.zeros((32,), jnp.bfloat16)
        zcnt_v[i, :] = jnp.zeros((16,), jnp.float32)
        return carry
    lax.fori_loop(0, ZROWS, fill_z, 0)

    def fill_o(i, carry):
        ones_v[i, :] = jnp.full((16,), 1.0, jnp.float32)
        return carry
    lax.fori_loop(0, CHUNK, fill_o, 0)

    pltpu.sync_copy(rows_hbm.at[s], ridx_v)
    pltpu.sync_copy(cols2_hbm.at[c, s], cidx_v)

    rbase = s * ROWS_PER_TILE
    for k in range(ROWS_PER_TILE // ZROWS):
        pltpu.sync_copy(zrow_v, ssum.at[pl.ds(rbase + k * ZROWS, ZROWS)])
        pltpu.sync_copy(zcnt_v, scnt.at[pl.ds(rbase + k * ZROWS, ZROWS)])
    plsc.subcore_barrier()

    xtab = xs_hbm
    for k in range(2):
        pltpu.async_copy(xtab.at[cidx_v.at[k]], feats[k], gsems[k])

    def step(i, carry):
        for j in range(NBUF):
            t = NBUF * i + j
            pltpu.make_async_copy(xtab.at[pl.ds(0, CHUNK)], feats[j],
                                  gsems[j]).wait()
            pltpu.async_copy(feats[j], ssum.at[ridx_v.at[t]], ssems[j],
                             add=True)

            @pl.when(t // CNT_STEPS == c)
            def _count():
                pltpu.async_copy(ones_v, scnt.at[ridx_v.at[t]], csem,
                                 add=True)

            tf = t + 2
            jf = (j + 2) % NBUF

            @pl.when(tf >= NBUF)
            def _reuse_wait():
                pltpu.make_async_copy(feats[jf], ssum.at[ridx_v.at[0]],
                                      ssems[jf]).wait()

            @pl.when(tf < STEPS)
            def _prefetch():
                pltpu.async_copy(xtab.at[cidx_v.at[tf]], feats[jf],
                                 gsems[jf])
        return carry
    lax.fori_loop(0, STEPS // NBUF, step, 0)

    for t in range(STEPS - NBUF + 2, STEPS):
        pltpu.make_async_copy(feats[t % NBUF], ssum.at[ridx_v.at[0]],
                              ssems[t % NBUF]).wait()

    def drain_cnt(i, carry):
        pltpu.make_async_copy(ones_v, scnt.at[ridx_v.at[0]], csem).wait()
        return carry
    lax.fori_loop(0, CNT_STEPS, drain_cnt, 0)

    plsc.subcore_barrier()

    @pl.when(c == 0)
    def _out0():
        pltpu.sync_copy(ssum.at[pl.ds(rbase, ROWS_PER_TILE)],
                        sum_hbm.at[pl.ds(rbase, ROWS_PER_TILE),
                                   pl.ds(0, DH)])
        pltpu.sync_copy(scnt.at[pl.ds(rbase, ROWS_PER_TILE)],
                        cnt0_hbm.at[pl.ds(rbase, ROWS_PER_TILE)])

    @pl.when(c == 1)
    def _out1():
        pltpu.sync_copy(ssum.at[pl.ds(rbase, ROWS_PER_TILE)],
                        sum_hbm.at[pl.ds(rbase, ROWS_PER_TILE),
                                   pl.ds(DH, DH)])
        pltpu.sync_copy(scnt.at[pl.ds(rbase, ROWS_PER_TILE)],
                        cnt1_hbm.at[pl.ds(rbase, ROWS_PER_TILE)])


def _tc_xw1(x_ref, w1_ref, b_ref, y_ref):
    y_ref[...] = jnp.dot(x_ref[...], w1_ref[...],
                         preferred_element_type=jnp.float32) + b_ref[...]


def _tc_tail(y_ref, s_ref, c0_ref, c1_ref, w2_ref, o_ref):
    cnt = c0_ref[...][:, 0:1] + c1_ref[...][:, 0:1]
    inv = 1.0 / (cnt + 1e-8)
    mean = s_ref[...].astype(jnp.float32) * inv
    acc = y_ref[...] + jnp.dot(mean, w2_ref[...],
                               preferred_element_type=jnp.float32)
    o_ref[...] = jnp.maximum(acc, 0.0)


@jax.jit
def kernel(x, edge_index, W, b):
    ei = edge_index.astype(jnp.int32)
    rows = ei[0].reshape(NS, STEPS, CHUNK)
    c2 = ei[1] * 2
    cols2 = jnp.stack([c2, c2 + 1]).reshape(NC, NS, STEPS, CHUNK)
    xs = x.astype(jnp.bfloat16).reshape(2 * N_NODES, DH)

    wt = W.T
    w1 = wt[:D]
    w2 = wt[D:]
    b2 = b.reshape(1, -1)

    blk = 1000
    y1 = pl.pallas_call(
        _tc_xw1,
        grid=(N_NODES // blk,),
        in_specs=[
            pl.BlockSpec((blk, D), lambda i: (i, 0)),
            pl.BlockSpec((D, D), lambda i: (0, 0)),
            pl.BlockSpec((1, D), lambda i: (0, 0)),
        ],
        out_specs=pl.BlockSpec((blk, D), lambda i: (i, 0)),
        out_shape=jax.ShapeDtypeStruct((N_NODES, D), jnp.float32),
    )(x, w1, b2)

    mesh = plsc.VectorSubcoreMesh(core_axis_name="c", subcore_axis_name="s")
    sc = pl.kernel(
        _sc_accumulate,
        out_type=(
            jax.ShapeDtypeStruct((NPAD, D), jnp.bfloat16),
            jax.ShapeDtypeStruct((NPAD, 16), jnp.float32),
            jax.ShapeDtypeStruct((NPAD, 16), jnp.float32),
        ),
        mesh=mesh,
        scratch_types=[
            pltpu.VMEM((STEPS, CHUNK), jnp.int32),
            pltpu.VMEM((STEPS, CHUNK), jnp.int32),
            [pltpu.VMEM((CHUNK, DH), jnp.bfloat16) for _ in range(NBUF)],
            pltpu.VMEM((CHUNK, 16), jnp.float32),
            pltpu.VMEM((ZROWS, DH), jnp.bfloat16),
            pltpu.VMEM((ZROWS, 16), jnp.float32),
            pltpu.VMEM_SHARED((NPAD, DH), jnp.bfloat16),
            pltpu.VMEM_SHARED((NPAD, 16), jnp.float32),
            [pltpu.SemaphoreType.DMA for _ in range(NBUF)],
            [pltpu.SemaphoreType.DMA for _ in range(NBUF)],
            pltpu.SemaphoreType.DMA,
        ],
        compiler_params=pltpu.CompilerParams(use_tc_tiling_on_sc=False),
    )
    sum_p, cnt0, cnt1 = sc(rows, cols2, xs)

    out = pl.pallas_call(
        _tc_tail,
        grid=(N_NODES // blk,),
        in_specs=[
            pl.BlockSpec((blk, D), lambda i: (i, 0)),
            pl.BlockSpec((blk, D), lambda i: (i, 0)),
            pl.BlockSpec((blk, 16), lambda i: (i, 0)),
            pl.BlockSpec((blk, 16), lambda i: (i, 0)),
            pl.BlockSpec((D, D), lambda i: (0, 0)),
        ],
        out_specs=pl.BlockSpec((blk, D), lambda i: (i, 0)),
        out_shape=jax.ShapeDtypeStruct((N_NODES, D), jnp.float32),
    )(y1, sum_p, cnt0, cnt1, w2)
    return out
